# baseline (speedup 1.0000x reference)
import jax
import jax.numpy as jnp
from jax import lax
from jax.experimental import pallas as pl
from jax.experimental.pallas import tpu as pltpu
from jax.experimental.pallas import tpu_sc as plsc

_R = 12800
_B = 4096
_BLK_R = 8
_NW = 32
_NBLK = (_R // _BLK_R) // _NW
_DEPTH = 3


def _sc_add(x2t, epat):
    mesh = plsc.VectorSubcoreMesh(core_axis_name="core", subcore_axis_name="subcore")

    @pl.kernel(
        mesh=mesh,
        out_type=jax.ShapeDtypeStruct((_R, _B), jnp.float32),
        scratch_types=[
            pltpu.VMEM((_DEPTH, _BLK_R, _B), jnp.float32),
            pltpu.VMEM((_DEPTH, _BLK_R, 128), jnp.float32),
            pltpu.SemaphoreType.DMA((_DEPTH,)),
            pltpu.SemaphoreType.DMA((_DEPTH,)),
            pltpu.SemaphoreType.DMA((_DEPTH,)),
        ],
    )
    def k(x_hbm, e_hbm, o_hbm, xb, eb, s_in, s_e, s_out):
        cid = lax.axis_index("core")
        sid = lax.axis_index("subcore")
        base = (sid * 2 + cid) * _NBLK

        def rsl(n):
            return (pl.ds(_BLK_R * (base + n), _BLK_R), pl.ds(0, _B))

        def esl(n):
            return (pl.ds(_BLK_R * (base + n), _BLK_R), pl.ds(0, 128))

        def start_in(n, p):
            pltpu.async_copy(x_hbm.at[rsl(n)], xb.at[p], s_in.at[p])
            pltpu.async_copy(e_hbm.at[esl(n)], eb.at[p], s_e.at[p])

        start_in(0, 0)

        @pl.loop(0, _NBLK, step=_DEPTH)
        def _(nn):
            for p in range(_DEPTH):
                n = nn + p

                @pl.when(n < _NBLK)
                def _(n=n, p=p):
                    q = (p + 1) % _DEPTH

                    @pl.when(n + 1 < _NBLK)
                    def _():
                        @pl.when(n >= 2)
                        def _():
                            pltpu.make_async_copy(xb.at[q], o_hbm.at[rsl(n - 2)], s_out.at[q]).wait()
                        start_in(n + 1, q)

                    pltpu.make_async_copy(x_hbm.at[rsl(n)], xb.at[p], s_in.at[p]).wait()
                    pltpu.make_async_copy(e_hbm.at[esl(n)], eb.at[p], s_e.at[p]).wait()

                    for r in range(_BLK_R):
                        evs = [eb.at[p, r, pl.ds(16 * kk, 16)][...] for kk in range(8)]

                        @plsc.parallel_loop(0, _B, step=128, unroll=2)
                        def _(g, evs=evs, r=r, p=p):
                            for kk in range(8):
                                plsc.addupdate(xb.at[p, r, pl.ds(g + 16 * kk, 16)], evs[kk])

                    pltpu.async_copy(xb.at[p], o_hbm.at[rsl(n)], s_out.at[p])

        for p in range(_DEPTH):
            n_last = _NBLK - _DEPTH + p
            pltpu.make_async_copy(
                xb.at[n_last % _DEPTH], o_hbm.at[rsl(n_last)], s_out.at[n_last % _DEPTH]
            ).wait()

    return k(x2t, epat)


def kernel(x, embedding):
    b, s, d = x.shape
    x2t = jnp.transpose(x, (1, 2, 0)).reshape(s * d, b)
    epat = jnp.broadcast_to(embedding.reshape(s * d, 1), (s * d, 128))
    out2 = _sc_add(x2t, epat)
    return jnp.transpose(out2.reshape(s, d, b), (2, 0, 1))


# speedup vs baseline: 1.0030x; 1.0030x over previous
import jax
import jax.numpy as jnp
from jax.experimental import pallas as pl
from jax.experimental.pallas import tpu as pltpu
from jax.experimental.pallas import tpu_sc as plsc

_R = 12800
_B = 4096
_BLK_R = 8
_BLK_B = 2048


def _sc_add(x2t, epat):
    mesh = plsc.VectorSubcoreMesh(core_axis_name="core", subcore_axis_name="subcore")

    @pl.kernel(
        mesh=mesh,
        out_type=jax.ShapeDtypeStruct((_R, _B), jnp.float32),
    )
    def k(x_hbm, e_hbm, o_hbm):
        def body(x_vmem, e_vmem, o_vmem):
            for r in range(_BLK_R):
                evs = [e_vmem.at[r, pl.ds(16 * kk, 16)][...] for kk in range(8)]

                @plsc.parallel_loop(0, _BLK_B, step=128, unroll=2)
                def _(g, evs=evs, r=r):
                    for kk in range(8):
                        sl = pl.ds(g + 16 * kk, 16)
                        o_vmem.at[r, sl][...] = x_vmem.at[r, sl][...] + evs[kk]

        pltpu.emit_pipeline(
            body,
            grid=(_R // _BLK_R, _B // _BLK_B),
            in_specs=[
                pl.BlockSpec((_BLK_R, _BLK_B), lambda i, j: (i, j)),
                pl.BlockSpec((_BLK_R, 128), lambda i, j: (i, 0)),
            ],
            out_specs=[pl.BlockSpec((_BLK_R, _BLK_B), lambda i, j: (i, j))],
            core_axis_name=("core", "subcore"),
            dimension_semantics=(pltpu.PARALLEL, pltpu.PARALLEL),
        )(x_hbm, e_hbm, o_hbm)

    return k(x2t, epat)


def kernel(x, embedding):
    b, s, d = x.shape
    x2t = jnp.transpose(x, (1, 2, 0)).reshape(s * d, b)
    epat = jnp.broadcast_to(embedding.reshape(s * d, 1), (s * d, 128))
    out2 = _sc_add(x2t, epat)
    return jnp.transpose(out2.reshape(s, d, b), (2, 0, 1))
